# baseline (speedup 1.0000x reference)
import functools

import jax
import jax.numpy as jnp
from jax import lax
from jax.experimental import pallas as pl
from jax.experimental.pallas import tpu as pltpu
from jax.experimental.pallas import tpu_sc as plsc

VOCAB = 1000000
EMB = 64
B = 4096
L = 200

NC = 2
NS = 16
NW = NC * NS
RPW = B // NW
NBUF = 4

_mesh = plsc.VectorSubcoreMesh(
    core_axis_name="c", subcore_axis_name="s", num_cores=NC, num_subcores=NS
)


@functools.partial(
    pl.kernel,
    out_type=jax.ShapeDtypeStruct((B, EMB), jnp.float32),
    mesh=_mesh,
    scratch_types=[
        pltpu.VMEM((L, RPW), jnp.int32),
        pltpu.VMEM((NBUF, RPW, EMB), jnp.float32),
        pltpu.VMEM((RPW, EMB), jnp.float32),
        [pltpu.SemaphoreType.DMA] * NBUF,
    ],
    compiler_params=pltpu.CompilerParams(use_tc_tiling_on_sc=False),
)
def _encode(ids_hbm, table_hbm, out_hbm, ids_v, emb_v, out_v, sems):
    wid = lax.axis_index("s") * NC + lax.axis_index("c")
    base = wid * RPW

    pltpu.sync_copy(ids_hbm.at[:, pl.ds(base, RPW)], ids_v)

    z = jnp.zeros((16,), jnp.float32)

    def zero_body(r, carry):
        for c in range(EMB // 16):
            out_v[r, pl.ds(16 * c, 16)] = z
        return carry

    lax.fori_loop(0, RPW, zero_body, 0)

    def fire(l, b):
        pltpu.async_copy(table_hbm.at[ids_v.at[l]], emb_v.at[b], sems[b])

    def drain(b):
        pltpu.make_async_copy(table_hbm.at[ids_v.at[0]], emb_v.at[b],
                              sems[b]).wait()

    def accumulate(b):
        def acc_body(i, carry):
            r = 2 * i
            for u in range(2):
                for c in range(EMB // 16):
                    plsc.addupdate(out_v.at[r + u, pl.ds(16 * c, 16)],
                                   emb_v[b, r + u, pl.ds(16 * c, 16)])
            return carry

        lax.fori_loop(0, RPW // 2, acc_body, 0)

    for b in range(NBUF):
        fire(b, b)

    def group_body(g, carry):
        l0 = NBUF * g
        for b in range(NBUF):
            drain(b)
            accumulate(b)
            fire(l0 + b + NBUF, b)
        return carry

    lax.fori_loop(0, L // NBUF - 1, group_body, 0)

    for b in range(NBUF):
        drain(b)
        accumulate(b)

    inv_l = jnp.full((16,), 1.0 / L, dtype=jnp.float32)

    def scale_body(r, carry):
        for c in range(EMB // 16):
            out_v[r, pl.ds(16 * c, 16)] = out_v[r, pl.ds(16 * c, 16)] * inv_l
        return carry

    lax.fori_loop(0, RPW, scale_body, 0)

    pltpu.sync_copy(out_v, out_hbm.at[pl.ds(base, RPW)])


def kernel(token_ids, table):
    return _encode(token_ids.T, table)
